# Initial kernel scaffold; baseline (speedup 1.0000x reference)
import jax
import jax.numpy as jnp
from jax.experimental import pallas as pl


def kernel(e0, e1, d0, d1):
    raise NotImplementedError("write your pallas kernel here")


# speedup vs baseline: 1.9516x; 1.9516x over previous
import numpy as np
import jax
import jax.numpy as jnp
from jax import lax
from jax.experimental import pallas as pl
from jax.experimental.pallas import tpu as pltpu


def _blend_1d(n: int) -> np.ndarray:
    o = np.arange(2 * n)
    src = (o + 0.5) / 2.0 - 0.5
    lo = np.floor(src)
    f = (src - lo).astype(np.float32)
    lo_i = np.clip(lo.astype(np.int64), 0, n - 1)
    hi_i = np.clip(lo.astype(np.int64) + 1, 0, n - 1)
    m = np.zeros((2 * n, n), np.float32)
    np.add.at(m, (o, lo_i), 1.0 - f)
    np.add.at(m, (o, hi_i), f)
    return m


def _up2x_block(x, aw):
    bb, H, W = x.shape
    y = lax.dot_general(
        x.reshape(bb * H, W), aw, (((1,), (0,)), ((), ())),
        preferred_element_type=jnp.float32,
    ).reshape(bb, H, 2 * W)
    if H > 1:
        prev = jnp.concatenate([y[:, :1], y[:, :-1]], axis=1)
        nxt = jnp.concatenate([y[:, 1:], y[:, -1:]], axis=1)
    else:
        prev = nxt = y
    pair = jnp.stack([0.75 * y + 0.25 * prev, 0.75 * y + 0.25 * nxt], axis=2)
    return pair.reshape(bb, 2 * H, 2 * W)


def _edup_body(aw_ref, mf_ref, e0_ref, d0_ref, e1_ref, d1_ref,
               ue0_ref, ud0_ref, ue1_ref, ud1_ref):
    aw = aw_ref[...]
    ue0_ref[...] = _up2x_block(e0_ref[...], aw).astype(ue0_ref.dtype)
    ud0_ref[...] = _up2x_block(d0_ref[...], aw).astype(ud0_ref.dtype)
    mf = mf_ref[...]
    ue1_ref[...] = jnp.dot(
        e1_ref[...], mf, preferred_element_type=jnp.float32
    ).astype(ue1_ref.dtype)
    ud1_ref[...] = jnp.dot(
        d1_ref[...], mf, preferred_element_type=jnp.float32
    ).astype(ud1_ref.dtype)


def kernel(e0, e1, d0, d1):
    N, C, H0, W0 = e0.shape
    _, _, H1, W1 = e1.shape
    B = N * C
    HW1 = H1 * W1

    G = min(32, B)
    while B % G:
        G //= 2
    bb = B // G

    aw = jnp.asarray(_blend_1d(W0).T)
    mf = jnp.asarray(np.kron(_blend_1d(H1), _blend_1d(W1)).T)

    itemsize = jnp.dtype(e0.dtype).itemsize
    out = pl.pallas_call(
        _edup_body,
        grid=(G,),
        in_specs=[
            pl.BlockSpec((W0, 2 * W0), lambda i: (0, 0)),
            pl.BlockSpec((HW1, 4 * HW1), lambda i: (0, 0)),
            pl.BlockSpec((bb, H0, W0), lambda i: (i, 0, 0)),
            pl.BlockSpec((bb, H0, W0), lambda i: (i, 0, 0)),
            pl.BlockSpec((bb, HW1), lambda i: (i, 0)),
            pl.BlockSpec((bb, HW1), lambda i: (i, 0)),
        ],
        out_specs=[
            pl.BlockSpec((bb, 2 * H0, 2 * W0), lambda i: (i, 0, 0)),
            pl.BlockSpec((bb, 2 * H0, 2 * W0), lambda i: (i, 0, 0)),
            pl.BlockSpec((bb, 4 * HW1), lambda i: (i, 0)),
            pl.BlockSpec((bb, 4 * HW1), lambda i: (i, 0)),
        ],
        out_shape=[
            jax.ShapeDtypeStruct((B, 2 * H0, 2 * W0), e0.dtype),
            jax.ShapeDtypeStruct((B, 2 * H0, 2 * W0), d0.dtype),
            jax.ShapeDtypeStruct((B, 4 * HW1), e1.dtype),
            jax.ShapeDtypeStruct((B, 4 * HW1), d1.dtype),
        ],
        compiler_params=pltpu.CompilerParams(
            dimension_semantics=("parallel",),
            vmem_limit_bytes=48 * 1024 * 1024,
        ),
        cost_estimate=pl.CostEstimate(
            flops=2 * 2 * B * (H0 * W0 * 2 * W0 + HW1 * 4 * HW1),
            transcendentals=0,
            bytes_accessed=5 * 2 * B * (H0 * W0 + HW1) * itemsize,
        ),
    )(aw, mf,
      e0.reshape(B, H0, W0), d0.reshape(B, H0, W0),
      e1.reshape(B, HW1), d1.reshape(B, HW1))

    ue0, ud0, ue1, ud1 = out
    return ([ue0.reshape(N, C, 2 * H0, 2 * W0),
             ue1.reshape(N, C, 2 * H1, 2 * W1)],
            [ud0.reshape(N, C, 2 * H0, 2 * W0),
             ud1.reshape(N, C, 2 * H1, 2 * W1)])


# speedup vs baseline: 3.1396x; 1.6087x over previous
import numpy as np
import jax
import jax.numpy as jnp
from jax import lax
from jax.experimental import pallas as pl
from jax.experimental.pallas import tpu as pltpu


def _blend_1d(n: int) -> np.ndarray:
    o = np.arange(2 * n)
    src = (o + 0.5) / 2.0 - 0.5
    lo = np.floor(src)
    f = (src - lo).astype(np.float32)
    lo_i = np.clip(lo.astype(np.int64), 0, n - 1)
    hi_i = np.clip(lo.astype(np.int64) + 1, 0, n - 1)
    m = np.zeros((2 * n, n), np.float32)
    np.add.at(m, (o, lo_i), 1.0 - f)
    np.add.at(m, (o, hi_i), f)
    return m


def _up2x_block(x, aw, ah):
    bb, H, W = x.shape
    y = lax.dot_general(
        x.reshape(bb * H, W), aw, (((1,), (0,)), ((), ())),
        preferred_element_type=jnp.float32,
    ).reshape(bb, H, 2 * W)
    z = lax.dot_general(
        jnp.broadcast_to(ah, (bb,) + ah.shape), y,
        (((2,), (1,)), ((0,), (0,))),
        preferred_element_type=jnp.float32,
    )
    return z


def _edup_body(aw_ref, ah_ref, mf_ref, e0_ref, d0_ref, e1_ref, d1_ref,
               ue0_ref, ud0_ref, ue1_ref, ud1_ref):
    aw = aw_ref[...]
    ah = ah_ref[...]
    ue0_ref[...] = _up2x_block(e0_ref[...], aw, ah).astype(ue0_ref.dtype)
    ud0_ref[...] = _up2x_block(d0_ref[...], aw, ah).astype(ud0_ref.dtype)
    mf = mf_ref[...]
    ue1_ref[...] = jnp.dot(
        e1_ref[...], mf, preferred_element_type=jnp.float32
    ).astype(ue1_ref.dtype)
    ud1_ref[...] = jnp.dot(
        d1_ref[...], mf, preferred_element_type=jnp.float32
    ).astype(ud1_ref.dtype)


def kernel(e0, e1, d0, d1):
    N, C, H0, W0 = e0.shape
    _, _, H1, W1 = e1.shape
    B = N * C
    HW1 = H1 * W1

    G = min(32, B)
    while B % G:
        G //= 2
    bb = B // G

    aw = jnp.asarray(_blend_1d(W0).T)
    ah = jnp.asarray(_blend_1d(H0))
    mf = jnp.asarray(np.kron(_blend_1d(H1), _blend_1d(W1)).T)

    itemsize = jnp.dtype(e0.dtype).itemsize
    out = pl.pallas_call(
        _edup_body,
        grid=(G,),
        in_specs=[
            pl.BlockSpec((W0, 2 * W0), lambda i: (0, 0)),
            pl.BlockSpec((2 * H0, H0), lambda i: (0, 0)),
            pl.BlockSpec((HW1, 4 * HW1), lambda i: (0, 0)),
            pl.BlockSpec((bb, H0, W0), lambda i: (i, 0, 0)),
            pl.BlockSpec((bb, H0, W0), lambda i: (i, 0, 0)),
            pl.BlockSpec((bb, HW1), lambda i: (i, 0)),
            pl.BlockSpec((bb, HW1), lambda i: (i, 0)),
        ],
        out_specs=[
            pl.BlockSpec((bb, 2 * H0, 2 * W0), lambda i: (i, 0, 0)),
            pl.BlockSpec((bb, 2 * H0, 2 * W0), lambda i: (i, 0, 0)),
            pl.BlockSpec((bb, 4 * HW1), lambda i: (i, 0)),
            pl.BlockSpec((bb, 4 * HW1), lambda i: (i, 0)),
        ],
        out_shape=[
            jax.ShapeDtypeStruct((B, 2 * H0, 2 * W0), e0.dtype),
            jax.ShapeDtypeStruct((B, 2 * H0, 2 * W0), d0.dtype),
            jax.ShapeDtypeStruct((B, 4 * HW1), e1.dtype),
            jax.ShapeDtypeStruct((B, 4 * HW1), d1.dtype),
        ],
        compiler_params=pltpu.CompilerParams(
            dimension_semantics=("parallel",),
            vmem_limit_bytes=48 * 1024 * 1024,
        ),
        cost_estimate=pl.CostEstimate(
            flops=2 * 2 * B * (H0 * W0 * 2 * W0 + HW1 * 4 * HW1),
            transcendentals=0,
            bytes_accessed=5 * 2 * B * (H0 * W0 + HW1) * itemsize,
        ),
    )(aw, ah, mf,
      e0.reshape(B, H0, W0), d0.reshape(B, H0, W0),
      e1.reshape(B, HW1), d1.reshape(B, HW1))

    ue0, ud0, ue1, ud1 = out
    return ([ue0.reshape(N, C, 2 * H0, 2 * W0),
             ue1.reshape(N, C, 2 * H1, 2 * W1)],
            [ud0.reshape(N, C, 2 * H0, 2 * W0),
             ud1.reshape(N, C, 2 * H1, 2 * W1)])
